# baseline (speedup 1.0000x reference)
import functools

import jax
import jax.numpy as jnp
from jax import lax
from jax.experimental import pallas as pl
from jax.experimental.pallas import tpu as pltpu
from jax.experimental.pallas import tpu_sc as plsc

N_CLS = 1000
N_CTX = 16
D = 512
LN = 128
SL = D // LN
SEQ = 77
TOK = SEQ - 1
HDR = 1 + N_CTX
ROWS = HDR + TOK
NC, NS = 2, 16
NW = NC * NS
CPW = 32


@functools.partial(
    pl.kernel,
    out_type=jax.ShapeDtypeStruct((N_CLS * ROWS, SL, LN), jnp.float32),
    mesh=plsc.VectorSubcoreMesh(
        core_axis_name="c", subcore_axis_name="s",
        num_cores=NC, num_subcores=NS,
    ),
    scratch_types=[
        pltpu.VMEM((CPW, TOK), jnp.int32),
        pltpu.VMEM((ROWS, SL, LN), jnp.float32),
        pltpu.VMEM((ROWS, SL, LN), jnp.float32),
        pltpu.SemaphoreType.DMA,
        pltpu.SemaphoreType.DMA,
        pltpu.SemaphoreType.DMA,
        pltpu.SemaphoreType.DMA,
    ],
    compiler_params=pltpu.CompilerParams(use_tc_tiling_on_sc=False),
)
def _prompt_assemble(idx_hbm, emb_hbm, ctx_hbm, pref_hbm, out_hbm,
                     idx_v, buf_a, buf_b,
                     sem_ain, sem_aout, sem_bin, sem_bout):
    wid = lax.axis_index("s") * NC + lax.axis_index("c")
    base = jnp.minimum(wid * CPW, N_CLS - CPW)

    bufs = ((buf_a, sem_ain, sem_aout), (buf_b, sem_bin, sem_bout))

    def gather_start(buf, t, sem):
        pltpu.make_async_copy(pref_hbm.at[pl.ds(base + t, 1)],
                              buf.at[pl.ds(0, 1)], sem).start()
        pltpu.make_async_copy(emb_hbm.at[idx_v.at[t]],
                              buf.at[pl.ds(HDR, TOK)], sem).start()

    def gather_wait(buf, sem):
        pltpu.make_async_copy(pref_hbm.at[pl.ds(0, 1)],
                              buf.at[pl.ds(0, 1)], sem).wait()
        pltpu.make_async_copy(emb_hbm.at[idx_v.at[0]],
                              buf.at[pl.ds(HDR, TOK)], sem).wait()

    def scatter_start(buf, t, sem):
        pltpu.make_async_copy(buf, out_hbm.at[pl.ds((base + t) * ROWS, ROWS)],
                              sem).start()

    def scatter_wait(buf, sem):
        pltpu.make_async_copy(buf, out_hbm.at[pl.ds(0, ROWS)], sem).wait()

    pltpu.sync_copy(idx_hbm.at[pl.ds(base, CPW)], idx_v)
    pltpu.sync_copy(ctx_hbm, buf_a.at[pl.ds(1, N_CTX)])
    pltpu.sync_copy(ctx_hbm, buf_b.at[pl.ds(1, N_CTX)])
    for k, (buf, sin, _) in enumerate(bufs):
        gather_start(buf, k, sin)

    def body(i, carry):
        for k, (buf, sin, sout) in enumerate(bufs):
            gather_wait(buf, sin)
            scatter_start(buf, 2 * i + k, sout)
        for k, (buf, sin, sout) in enumerate(bufs):
            scatter_wait(buf, sout)
            gather_start(buf, 2 * i + k + 2, sin)
        return carry

    lax.fori_loop(0, CPW // 2 - 1, body, 0)

    for k, (buf, sin, sout) in enumerate(bufs):
        gather_wait(buf, sin)
        scatter_start(buf, CPW - 2 + k, sout)
    for _, (buf, _, sout) in enumerate(bufs):
        scatter_wait(buf, sout)


def kernel(tokenized_text, token_embedding, ctx, token_prefix):
    vocab = token_embedding.shape[0]
    idx = tokenized_text[:, 1:].astype(jnp.int32)
    emb3 = token_embedding.reshape(vocab, SL, LN)
    ctx3 = ctx[0].reshape(N_CTX, SL, LN)
    pref3 = token_prefix.reshape(N_CLS, SL, LN)
    out = _prompt_assemble(idx, emb3, ctx3, pref3)
    return out.reshape(N_CLS, ROWS, D)


# speedup vs baseline: 3.2802x; 3.2802x over previous
import functools

import jax
import jax.numpy as jnp
from jax import lax
from jax.experimental import pallas as pl
from jax.experimental.pallas import tpu as pltpu
from jax.experimental.pallas import tpu_sc as plsc

N_CLS = 1000
N_CTX = 16
D = 512
LN = 128
SL = D // LN
SEQ = 77
TOK = SEQ - 1
HDR = 1 + N_CTX
ROWS = HDR + TOK
CB = N_CLS // 8
SEG = CB
HSEG = HDR * SL * 8
GSEG = TOK * SL * 8
NC, NS = 2, 16
NW = NC * NS
HPW = HSEG // NW
GPW = GSEG // NW
NBUF = 4


@functools.partial(
    pl.kernel,
    out_type=jax.ShapeDtypeStruct((ROWS * N_CLS * SL, LN), jnp.float32),
    mesh=plsc.VectorSubcoreMesh(
        core_axis_name="c", subcore_axis_name="s",
        num_cores=NC, num_subcores=NS,
    ),
    scratch_types=[
        pltpu.VMEM((HPW, SEG), jnp.int32),
        pltpu.VMEM((GPW, SEG), jnp.int32),
        [pltpu.VMEM((SEG, LN), jnp.float32) for _ in range(NBUF)],
        [pltpu.SemaphoreType.DMA for _ in range(NBUF)],
        [pltpu.SemaphoreType.DMA for _ in range(NBUF)],
    ],
    compiler_params=pltpu.CompilerParams(use_tc_tiling_on_sc=False),
)
def _prompt_assemble(hidx_hbm, gidx_hbm, hsrc_hbm, emb_hbm, out_hbm,
                     hidx_v, gidx_v, bufs, sem_in, sem_out):
    wid = lax.axis_index("s") * NC + lax.axis_index("c")
    hbase = wid * HPW
    gbase = wid * GPW

    def gstart(b, src, idx_row):
        pltpu.make_async_copy(src.at[idx_row], bufs[b], sem_in[b]).start()

    def gwait(b, src, slab):
        pltpu.make_async_copy(src.at[slab.at[0]], bufs[b], sem_in[b]).wait()

    def sstart(b, seg):
        pltpu.make_async_copy(bufs[b], out_hbm.at[pl.ds(seg * SEG, SEG)],
                              sem_out[b]).start()

    def swait(b):
        pltpu.make_async_copy(bufs[b], out_hbm.at[pl.ds(0, SEG)],
                              sem_out[b]).wait()

    pltpu.sync_copy(hidx_hbm.at[pl.ds(hbase, HPW)], hidx_v)
    pltpu.sync_copy(gidx_hbm.at[pl.ds(gbase, GPW)], gidx_v)

    for b in range(NBUF):
        gstart(b, hsrc_hbm, hidx_v.at[b])
    for b in range(NBUF):
        gwait(b, hsrc_hbm, hidx_v)
        sstart(b, hbase + b)

    def hbody(g, carry):
        for b in range(NBUF):
            swait(b)
            gstart(b, hsrc_hbm, hidx_v.at[NBUF * g + b])
        for b in range(NBUF):
            gwait(b, hsrc_hbm, hidx_v)
            sstart(b, hbase + NBUF * g + b)
        return carry

    lax.fori_loop(1, HPW // NBUF, hbody, 0)

    swait(0)
    gstart(0, hsrc_hbm, hidx_v.at[HPW - 1])
    gwait(0, hsrc_hbm, hidx_v)
    sstart(0, hbase + HPW - 1)

    def gbody(g, carry):
        for b in range(NBUF):
            swait(b)
            gstart(b, emb_hbm, gidx_v.at[NBUF * g + b])
        for b in range(NBUF):
            gwait(b, emb_hbm, gidx_v)
            sstart(b, HSEG + gbase + NBUF * g + b)
        return carry

    lax.fori_loop(0, GPW // NBUF, gbody, 0)

    for b in range(NBUF):
        swait(b)


def kernel(tokenized_text, token_embedding, ctx, token_prefix):
    vocab = token_embedding.shape[0]
    i4 = jnp.arange(SL, dtype=jnp.int32)

    t = tokenized_text[:, 1:].astype(jnp.int32)
    tb = t.T.reshape(TOK, CB, 1, 8)
    gidx = (tb * SL + i4.reshape(1, 1, SL, 1)).reshape(GSEG, SEG)

    hsrc = jnp.concatenate([ctx[0].reshape(N_CTX * SL, LN),
                            token_prefix.reshape(N_CLS * SL, LN)], axis=0)
    c_ids = jnp.arange(N_CLS, dtype=jnp.int32).reshape(CB, 1, 8)
    p0 = N_CTX * SL + SL * c_ids + i4.reshape(1, SL, 1)
    j_ids = jnp.arange(N_CTX, dtype=jnp.int32).reshape(N_CTX, 1, 1, 1)
    pj = jnp.broadcast_to(SL * j_ids + i4.reshape(1, 1, SL, 1),
                          (N_CTX, CB, SL, 8))
    hidx = jnp.concatenate([p0.reshape(1, CB, SL, 8), pj],
                           axis=0).reshape(HSEG, SEG)

    emb2 = token_embedding.reshape(vocab * SL, LN)
    out = _prompt_assemble(hidx, gidx, hsrc, emb2)
    return (out.reshape(ROWS, CB, SL, 8, LN)
               .transpose(1, 3, 0, 2, 4)
               .reshape(N_CLS, ROWS, D))
